# baseline (speedup 1.0000x reference)
import functools

import jax
import jax.numpy as jnp
from jax import lax
from jax.experimental import pallas as pl
from jax.experimental.pallas import tpu as pltpu
from jax.experimental.pallas import tpu_sc as plsc

L = 16
NW = 32
NC = 2


def _sc_pose_correction(n_rows, batch):
    b_per_w = batch // NW
    groups = b_per_w // L
    mesh = plsc.VectorSubcoreMesh(core_axis_name="c", subcore_axis_name="s")

    @functools.partial(
        pl.kernel,
        mesh=mesh,
        compiler_params=pltpu.CompilerParams(needs_layout_passes=False),
        out_type=jax.ShapeDtypeStruct((6, batch), jnp.float32),
        scratch_types=[
            pltpu.VMEM((7, n_rows), jnp.float32),
            pltpu.VMEM((b_per_w,), jnp.int32),
            pltpu.VMEM((b_per_w,), jnp.int32),
            pltpu.VMEM((6, b_per_w), jnp.float32),
            pltpu.VMEM((6, b_per_w), jnp.float32),
            pltpu.SemaphoreType.DMA,
            pltpu.SemaphoreType.DMA,
            pltpu.SemaphoreType.DMA,
            pltpu.SemaphoreType.DMA,
        ],
    )
    def k(table_hbm, idx_hbm, mask_hbm, rays_hbm, out_hbm,
          table_v, idx_v, mask_v, rays_v, out_v,
          sem_t, sem_i, sem_m, sem_r):
        wid = lax.axis_index("s") * NC + lax.axis_index("c")
        base = wid * b_per_w
        cps = [pltpu.async_copy(table_hbm, table_v, sem_t)]
        cps.append(pltpu.async_copy(
            idx_hbm.at[pl.ds(base, b_per_w)], idx_v, sem_i))
        cps.append(pltpu.async_copy(
            mask_hbm.at[pl.ds(base, b_per_w)], mask_v, sem_m))
        cps.append(pltpu.async_copy(
            rays_hbm.at[:, pl.ds(base, b_per_w)], rays_v, sem_r))
        for cp in cps:
            cp.wait()

        zeros = jnp.zeros((L,), jnp.float32)
        ones = jnp.ones((L,), jnp.float32)

        @plsc.parallel_loop(0, groups, unroll=4)
        def body(g):
            sl = pl.ds(g * L, L)
            idx = idx_v[sl]
            m = mask_v[sl] == 1

            def gat(c, ident):
                col = jnp.full((L,), c, jnp.int32)
                return jnp.where(m, plsc.load_gather(table_v, [col, idx]), ident)

            tx = gat(0, zeros)
            ty = gat(1, zeros)
            tz = gat(2, zeros)
            qx = gat(3, zeros)
            qy = gat(4, zeros)
            qz = gat(5, zeros)
            qw = gat(6, ones)

            dx = rays_v[3, sl]
            dy = rays_v[4, sl]
            dz = rays_v[5, sl]

            r00 = 1.0 - 2.0 * (qy * qy + qz * qz)
            r01 = 2.0 * (qx * qy - qz * qw)
            r02 = 2.0 * (qx * qz + qy * qw)
            r10 = 2.0 * (qx * qy + qz * qw)
            r11 = 1.0 - 2.0 * (qx * qx + qz * qz)
            r12 = 2.0 * (qy * qz - qx * qw)
            r20 = 2.0 * (qx * qz - qy * qw)
            r21 = 2.0 * (qy * qz + qx * qw)
            r22 = 1.0 - 2.0 * (qx * qx + qy * qy)

            out_v[0, sl] = rays_v[0, sl] + tx
            out_v[1, sl] = rays_v[1, sl] + ty
            out_v[2, sl] = rays_v[2, sl] + tz
            out_v[3, sl] = r00 * dx + r01 * dy + r02 * dz
            out_v[4, sl] = r10 * dx + r11 * dy + r12 * dz
            out_v[5, sl] = r20 * dx + r21 * dy + r22 * dz

        pltpu.sync_copy(out_v, out_hbm.at[:, pl.ds(base, b_per_w)])

    return k


def kernel(image_indices, rays, depth_mask, correction_dict):
    batch = rays.shape[0]
    n_rows = correction_dict.shape[0]

    table_t = correction_dict.T
    idx = image_indices.astype(jnp.int32)
    mask = depth_mask.astype(jnp.int32).reshape(-1)
    rays_t = rays.T

    out = _sc_pose_correction(n_rows, batch)(table_t, idx, mask, rays_t)
    return out.T


# speedup vs baseline: 1.0023x; 1.0023x over previous
import functools

import jax
import jax.numpy as jnp
from jax import lax
from jax.experimental import pallas as pl
from jax.experimental.pallas import tpu as pltpu
from jax.experimental.pallas import tpu_sc as plsc

L = 16
NW = 32
NC = 2


def _sc_pose_correction(n_rows, batch):
    b_per_w = batch // NW
    groups = b_per_w // L
    mesh = plsc.VectorSubcoreMesh(core_axis_name="c", subcore_axis_name="s")

    @functools.partial(
        pl.kernel,
        mesh=mesh,
        compiler_params=pltpu.CompilerParams(
            needs_layout_passes=False,
            disable_bounds_checks=True,
            disable_semaphore_checks=True,
            skip_device_barrier=True,
        ),
        out_type=jax.ShapeDtypeStruct((6, batch), jnp.float32),
        scratch_types=[
            pltpu.VMEM((7, n_rows), jnp.float32),
            pltpu.VMEM((b_per_w,), jnp.int32),
            pltpu.VMEM((b_per_w,), jnp.int32),
            pltpu.VMEM((6, b_per_w), jnp.float32),
            pltpu.VMEM((6, b_per_w), jnp.float32),
            pltpu.SemaphoreType.DMA,
            pltpu.SemaphoreType.DMA,
            pltpu.SemaphoreType.DMA,
            pltpu.SemaphoreType.DMA,
        ],
    )
    def k(table_hbm, idx_hbm, mask_hbm, rays_hbm, out_hbm,
          table_v, idx_v, mask_v, rays_v, out_v,
          sem_t, sem_i, sem_m, sem_r):
        wid = lax.axis_index("s") * NC + lax.axis_index("c")
        base = wid * b_per_w
        cps = [pltpu.async_copy(table_hbm, table_v, sem_t)]
        cps.append(pltpu.async_copy(
            idx_hbm.at[pl.ds(base, b_per_w)], idx_v, sem_i))
        cps.append(pltpu.async_copy(
            mask_hbm.at[pl.ds(base, b_per_w)], mask_v, sem_m))
        cps.append(pltpu.async_copy(
            rays_hbm.at[:, pl.ds(base, b_per_w)], rays_v, sem_r))
        for cp in cps:
            cp.wait()

        zeros = jnp.zeros((L,), jnp.float32)
        ones = jnp.ones((L,), jnp.float32)

        @plsc.parallel_loop(0, groups, unroll=4)
        def body(g):
            sl = pl.ds(g * L, L)
            idx = idx_v[sl]
            m = mask_v[sl] == 1

            def gat(c, ident):
                col = jnp.full((L,), c, jnp.int32)
                return jnp.where(m, plsc.load_gather(table_v, [col, idx]), ident)

            tx = gat(0, zeros)
            ty = gat(1, zeros)
            tz = gat(2, zeros)
            qx = gat(3, zeros)
            qy = gat(4, zeros)
            qz = gat(5, zeros)
            qw = gat(6, ones)

            dx = rays_v[3, sl]
            dy = rays_v[4, sl]
            dz = rays_v[5, sl]

            r00 = 1.0 - 2.0 * (qy * qy + qz * qz)
            r01 = 2.0 * (qx * qy - qz * qw)
            r02 = 2.0 * (qx * qz + qy * qw)
            r10 = 2.0 * (qx * qy + qz * qw)
            r11 = 1.0 - 2.0 * (qx * qx + qz * qz)
            r12 = 2.0 * (qy * qz - qx * qw)
            r20 = 2.0 * (qx * qz - qy * qw)
            r21 = 2.0 * (qy * qz + qx * qw)
            r22 = 1.0 - 2.0 * (qx * qx + qy * qy)

            out_v[0, sl] = rays_v[0, sl] + tx
            out_v[1, sl] = rays_v[1, sl] + ty
            out_v[2, sl] = rays_v[2, sl] + tz
            out_v[3, sl] = r00 * dx + r01 * dy + r02 * dz
            out_v[4, sl] = r10 * dx + r11 * dy + r12 * dz
            out_v[5, sl] = r20 * dx + r21 * dy + r22 * dz

        pltpu.sync_copy(out_v, out_hbm.at[:, pl.ds(base, b_per_w)])

    return k


def kernel(image_indices, rays, depth_mask, correction_dict):
    batch = rays.shape[0]
    n_rows = correction_dict.shape[0]

    table_t = correction_dict.T
    idx = image_indices.astype(jnp.int32)
    mask = depth_mask.astype(jnp.int32).reshape(-1)
    rays_t = rays.T

    out = _sc_pose_correction(n_rows, batch)(table_t, idx, mask, rays_t)
    return out.T


# speedup vs baseline: 1.0126x; 1.0103x over previous
import functools

import jax
import jax.numpy as jnp
from jax import lax
from jax.experimental import pallas as pl
from jax.experimental.pallas import tpu as pltpu
from jax.experimental.pallas import tpu_sc as plsc

L = 16
NW = 32
NC = 2


def _sc_pose_correction(n_rows, batch):
    b_per_w = batch // NW
    groups = b_per_w // L
    mesh = plsc.VectorSubcoreMesh(core_axis_name="c", subcore_axis_name="s")

    @functools.partial(
        pl.kernel,
        mesh=mesh,
        compiler_params=pltpu.CompilerParams(
            needs_layout_passes=False,
            disable_bounds_checks=True,
            disable_semaphore_checks=True,
            skip_device_barrier=True,
        ),
        out_type=jax.ShapeDtypeStruct((6, batch), jnp.float32),
        scratch_types=[
            pltpu.VMEM((7, n_rows), jnp.float32),
            pltpu.VMEM((b_per_w,), jnp.int32),
            pltpu.VMEM((b_per_w,), jnp.int32),
            pltpu.VMEM((6, b_per_w), jnp.float32),
            pltpu.VMEM((6, b_per_w), jnp.float32),
            pltpu.SemaphoreType.DMA,
            pltpu.SemaphoreType.DMA,
            pltpu.SemaphoreType.DMA,
            pltpu.SemaphoreType.DMA,
        ],
    )
    def k(table_hbm, idx_hbm, mask_hbm, rays_hbm, out_hbm,
          table_v, idx_v, mask_v, rays_v, out_v,
          sem_t, sem_i, sem_m, sem_r):
        wid = lax.axis_index("s") * NC + lax.axis_index("c")
        base = wid * b_per_w
        cps = [pltpu.async_copy(table_hbm, table_v, sem_t)]
        cps.append(pltpu.async_copy(
            idx_hbm.at[pl.ds(base, b_per_w)], idx_v, sem_i))
        cps.append(pltpu.async_copy(
            mask_hbm.at[pl.ds(base, b_per_w)], mask_v, sem_m))
        cps.append(pltpu.async_copy(
            rays_hbm.at[:, pl.ds(base, b_per_w)], rays_v, sem_r))
        for cp in cps:
            cp.wait()

        zeros = jnp.zeros((L,), jnp.float32)
        ones = jnp.ones((L,), jnp.float32)

        @plsc.parallel_loop(0, groups, unroll=1)
        def body(g):
            sl = pl.ds(g * L, L)
            idx = idx_v[sl]
            m = mask_v[sl] == 1

            def gat(c, ident):
                col = jnp.full((L,), c, jnp.int32)
                return jnp.where(m, plsc.load_gather(table_v, [col, idx]), ident)

            tx = gat(0, zeros)
            ty = gat(1, zeros)
            tz = gat(2, zeros)
            qx = gat(3, zeros)
            qy = gat(4, zeros)
            qz = gat(5, zeros)
            qw = gat(6, ones)

            dx = rays_v[3, sl]
            dy = rays_v[4, sl]
            dz = rays_v[5, sl]

            c1x = qy * dz - qz * dy
            c1y = qz * dx - qx * dz
            c1z = qx * dy - qy * dx
            c2x = qw * c1x + (qy * c1z - qz * c1y)
            c2y = qw * c1y + (qz * c1x - qx * c1z)
            c2z = qw * c1z + (qx * c1y - qy * c1x)

            out_v[0, sl] = rays_v[0, sl] + tx
            out_v[1, sl] = rays_v[1, sl] + ty
            out_v[2, sl] = rays_v[2, sl] + tz
            out_v[3, sl] = dx + 2.0 * c2x
            out_v[4, sl] = dy + 2.0 * c2y
            out_v[5, sl] = dz + 2.0 * c2z

        pltpu.sync_copy(out_v, out_hbm.at[:, pl.ds(base, b_per_w)])

    return k


def kernel(image_indices, rays, depth_mask, correction_dict):
    batch = rays.shape[0]
    n_rows = correction_dict.shape[0]

    table_t = correction_dict.T
    idx = image_indices.astype(jnp.int32)
    mask = depth_mask.astype(jnp.int32).reshape(-1)
    rays_t = rays.T

    out = _sc_pose_correction(n_rows, batch)(table_t, idx, mask, rays_t)
    return out.T


# speedup vs baseline: 1.0150x; 1.0023x over previous
import functools

import jax
import jax.numpy as jnp
from jax import lax
from jax.experimental import pallas as pl
from jax.experimental.pallas import tpu as pltpu
from jax.experimental.pallas import tpu_sc as plsc

L = 16
NW = 32
NC = 2


def _sc_pose_correction(n_rows, batch):
    b_per_w = batch // NW
    groups = b_per_w // L
    mesh = plsc.VectorSubcoreMesh(core_axis_name="c", subcore_axis_name="s")

    @functools.partial(
        pl.kernel,
        mesh=mesh,
        compiler_params=pltpu.CompilerParams(
            needs_layout_passes=False,
            disable_bounds_checks=True,
            disable_semaphore_checks=True,
            skip_device_barrier=True,
        ),
        out_type=jax.ShapeDtypeStruct((6, batch), jnp.float32),
        scratch_types=[
            pltpu.VMEM((7, n_rows), jnp.float32),
            pltpu.VMEM((b_per_w,), jnp.int32),
            pltpu.VMEM((b_per_w,), jnp.int32),
            pltpu.VMEM((6, b_per_w), jnp.float32),
            pltpu.VMEM((6, b_per_w), jnp.float32),
            pltpu.SemaphoreType.DMA,
            pltpu.SemaphoreType.DMA,
            pltpu.SemaphoreType.DMA,
            pltpu.SemaphoreType.DMA,
        ],
    )
    def k(table_hbm, idx_hbm, mask_hbm, rays_hbm, out_hbm,
          table_v, idx_v, mask_v, rays_v, out_v,
          sem_t, sem_i, sem_m, sem_r):
        wid = lax.axis_index("s") * NC + lax.axis_index("c")
        base = wid * b_per_w
        cps = [pltpu.async_copy(table_hbm, table_v, sem_t)]
        cps.append(pltpu.async_copy(
            idx_hbm.at[pl.ds(base, b_per_w)], idx_v, sem_i))
        cps.append(pltpu.async_copy(
            mask_hbm.at[pl.ds(base, b_per_w)], mask_v, sem_m))
        cps.append(pltpu.async_copy(
            rays_hbm.at[:, pl.ds(base, b_per_w)], rays_v, sem_r))
        for cp in cps:
            cp.wait()

        zeros = jnp.zeros((L,), jnp.float32)
        ones = jnp.ones((L,), jnp.float32)

        @plsc.parallel_loop(0, groups, unroll=2)
        def body(g):
            sl = pl.ds(g * L, L)
            idx = idx_v[sl]
            m = mask_v[sl] == 1

            def gat(c, ident):
                col = jnp.full((L,), c, jnp.int32)
                return jnp.where(m, plsc.load_gather(table_v, [col, idx]), ident)

            tx = gat(0, zeros)
            ty = gat(1, zeros)
            tz = gat(2, zeros)
            qx = gat(3, zeros)
            qy = gat(4, zeros)
            qz = gat(5, zeros)
            qw = gat(6, ones)

            dx = rays_v[3, sl]
            dy = rays_v[4, sl]
            dz = rays_v[5, sl]

            c1x = qy * dz - qz * dy
            c1y = qz * dx - qx * dz
            c1z = qx * dy - qy * dx
            c2x = qw * c1x + (qy * c1z - qz * c1y)
            c2y = qw * c1y + (qz * c1x - qx * c1z)
            c2z = qw * c1z + (qx * c1y - qy * c1x)

            out_v[0, sl] = rays_v[0, sl] + tx
            out_v[1, sl] = rays_v[1, sl] + ty
            out_v[2, sl] = rays_v[2, sl] + tz
            out_v[3, sl] = dx + 2.0 * c2x
            out_v[4, sl] = dy + 2.0 * c2y
            out_v[5, sl] = dz + 2.0 * c2z

        pltpu.sync_copy(out_v, out_hbm.at[:, pl.ds(base, b_per_w)])

    return k


def kernel(image_indices, rays, depth_mask, correction_dict):
    batch = rays.shape[0]
    n_rows = correction_dict.shape[0]

    table_t = correction_dict.T
    idx = image_indices.astype(jnp.int32)
    mask = depth_mask.astype(jnp.int32).reshape(-1)
    rays_t = rays.T

    out = _sc_pose_correction(n_rows, batch)(table_t, idx, mask, rays_t)
    return out.T
